# baseline (speedup 1.0000x reference)
import functools

import jax
import jax.numpy as jnp
from jax import lax
from jax.experimental import pallas as pl
from jax.experimental.pallas import tpu as pltpu
from jax.experimental.pallas import tpu_sc as plsc

N = 10000
E = 320000
D = 128
H = 8
DH = D // H
NC = 40

BLK = 1000
NGRID = N // BLK
NCORES = 2
NSUB = 16
NW = NCORES * NSUB
EW = E // NW
EB = 80
NCHUNK = EW // EB
NP = 10240
RPT = NP // NSUB


def _attn_tail(x, w_ref, ssd_ref, o_h, o_asad, o_m, ms):
    i = pl.program_id(0)
    h = jnp.dot(x, w_ref[...])
    o_h[...] = h
    asad = jnp.dot(h, ssd_ref[...])
    o_asad[...] = asad
    mx = jnp.max(asad, axis=0)

    @pl.when(i == 0)
    def _():
        ms[...] = jnp.full((8, 32), -1e30, jnp.float32)

    ms[...] = jnp.maximum(ms[...], mx[None, :])

    @pl.when(i == NGRID - 1)
    def _():
        m = ms[...]
        o_m[...] = jnp.maximum(m[:, :16] + m[:, 16:], 0.0)


def _combine(accp_ref, denp_ref, p16_ref):
    acc = accp_ref[0] + accp_ref[1]
    den = denp_ref[0] + denp_ref[1]
    rden = 1.0 / (den + 1e-16)
    rd = jnp.dot(rden, p16_ref[...])
    x = acc * rd
    return jnp.where(x > 0.0, x, jnp.exp(jnp.minimum(x, 0.0)) - 1.0)


def _prep0_body(x_ref, w_ref, ssd_ref, o_h, o_asad, o_m, ms):
    _attn_tail(x_ref[...], w_ref, ssd_ref, o_h, o_asad, o_m, ms)


def _mid_body(accp_ref, denp_ref, p16_ref, w_ref, ssd_ref,
              o_h, o_asad, o_m, ms):
    x = _combine(accp_ref, denp_ref, p16_ref)
    _attn_tail(x, w_ref, ssd_ref, o_h, o_asad, o_m, ms)


def _final_body(accp_ref, denp_ref, p16_ref, w1, b1, w2, b2, w3, b3, o_y):
    x = _combine(accp_ref, denp_ref, p16_ref)
    hdd = jnp.maximum(jnp.dot(x, w1[...]) + b1[...][None, :], 0.0)
    hdd = jnp.maximum(jnp.dot(hdd, w2[...]) + b2[...][None, :], 0.0)
    o_y[...] = jnp.dot(hdd, w3[...]) + b3[...][None, :]


def _prep0(X, W, ssd):
    return pl.pallas_call(
        _prep0_body,
        grid=(NGRID,),
        in_specs=[
            pl.BlockSpec((BLK, D), lambda i: (i, 0)),
            pl.BlockSpec((D, D), lambda i: (0, 0)),
            pl.BlockSpec((D, 32), lambda i: (0, 0)),
        ],
        out_specs=[
            pl.BlockSpec((BLK, D), lambda i: (i, 0)),
            pl.BlockSpec((BLK, 32), lambda i: (i, 0)),
            pl.BlockSpec((8, 16), lambda i: (0, 0)),
        ],
        out_shape=[
            jax.ShapeDtypeStruct((N, D), jnp.float32),
            jax.ShapeDtypeStruct((N, 32), jnp.float32),
            jax.ShapeDtypeStruct((8, 16), jnp.float32),
        ],
        scratch_shapes=[
            pltpu.VMEM((8, 32), jnp.float32),
        ],
    )(X, W, ssd)


def _mid(accp, denp, p16, W, ssd):
    return pl.pallas_call(
        _mid_body,
        grid=(NGRID,),
        in_specs=[
            pl.BlockSpec((2, BLK, D), lambda i: (0, i, 0)),
            pl.BlockSpec((2, BLK, 16), lambda i: (0, i, 0)),
            pl.BlockSpec((16, D), lambda i: (0, 0)),
            pl.BlockSpec((D, D), lambda i: (0, 0)),
            pl.BlockSpec((D, 32), lambda i: (0, 0)),
        ],
        out_specs=[
            pl.BlockSpec((BLK, D), lambda i: (i, 0)),
            pl.BlockSpec((BLK, 32), lambda i: (i, 0)),
            pl.BlockSpec((8, 16), lambda i: (0, 0)),
        ],
        out_shape=[
            jax.ShapeDtypeStruct((N, D), jnp.float32),
            jax.ShapeDtypeStruct((N, 32), jnp.float32),
            jax.ShapeDtypeStruct((8, 16), jnp.float32),
        ],
        scratch_shapes=[
            pltpu.VMEM((8, 32), jnp.float32),
        ],
    )(accp, denp, p16, W, ssd)


def _final(accp, denp, p16, Wf1, bf1, Wf2, bf2, Wf3, bf3):
    return pl.pallas_call(
        _final_body,
        grid=(NGRID,),
        in_specs=[
            pl.BlockSpec((2, BLK, D), lambda i: (0, i, 0)),
            pl.BlockSpec((2, BLK, 16), lambda i: (0, i, 0)),
            pl.BlockSpec((16, D), lambda i: (0, 0)),
            pl.BlockSpec((D, 512), lambda i: (0, 0)),
            pl.BlockSpec((512,), lambda i: (0,)),
            pl.BlockSpec((512, 512), lambda i: (0, 0)),
            pl.BlockSpec((512,), lambda i: (0,)),
            pl.BlockSpec((512, NC), lambda i: (0, 0)),
            pl.BlockSpec((NC,), lambda i: (0,)),
        ],
        out_specs=pl.BlockSpec((BLK, NC), lambda i: (i, 0)),
        out_shape=jax.ShapeDtypeStruct((N, NC), jnp.float32),
    )(accp, denp, p16, Wf1, bf1, Wf2, bf2, Wf3, bf3)


_GDN = lax.GatherDimensionNumbers(
    offset_dims=(), collapsed_slice_dims=(0,), start_index_map=(0,))

def _edge_sc(h, asad_t, edge_index, m_arr):
    mesh = plsc.VectorSubcoreMesh(core_axis_name="c", subcore_axis_name="s")

    @functools.partial(
        pl.kernel,
        mesh=mesh,
        compiler_params=pltpu.CompilerParams(use_tc_tiling_on_sc=False),
        out_type=[
            jax.ShapeDtypeStruct((NCORES, NP, D), jnp.float32),
            jax.ShapeDtypeStruct((NCORES, NP, 16), jnp.float32),
        ],
        scratch_types=[
            pltpu.VMEM_SHARED((NP, D), jnp.float32),
            pltpu.VMEM_SHARED((NP, 16), jnp.float32),
            pltpu.VMEM((2, EB), jnp.int32),
            pltpu.VMEM((2, EB), jnp.int32),
            pltpu.VMEM((EB, 32), jnp.float32),
            pltpu.VMEM((EB, 32), jnp.float32),
            pltpu.VMEM((EB, 32), jnp.float32),
            pltpu.VMEM((EB, 32), jnp.float32),
            pltpu.VMEM((EB, 16), jnp.float32),
            pltpu.VMEM((EB, 16), jnp.float32),
            pltpu.VMEM((EB, D), jnp.float32),
            pltpu.VMEM((EB, D), jnp.float32),
            pltpu.VMEM((EB,), jnp.int32),
            pltpu.VMEM((EB,), jnp.int32),
            pltpu.VMEM((16,), jnp.float32),
        ] + [pltpu.SemaphoreType.DMA] * 14,
    )
    def k(h_hbm, asad_hbm, src_hbm, dst_hbm, m_hbm, acc_out, den_out,
          acc_sh, den_sh, idxb0, idxb1, asb0, asb1, adb0, adb1,
          exb0, exb1, hb0, hb1, sdix0, sdix1, m_v,
          is0, is1, id0, id1, ga0, ga1, gb0, gb1, gh0, gh1,
          se0, se1, sa0, sa1):
        cid = lax.axis_index("c")
        sid = lax.axis_index("s")
        wid = cid * NSUB + sid
        zero16 = jnp.zeros((16,), jnp.float32)
        idxb = [idxb0, idxb1]
        asb = [asb0, asb1]
        adb = [adb0, adb1]
        exb = [exb0, exb1]
        hb = [hb0, hb1]
        sdix = [sdix0, sdix1]
        isem_s = [is0, is1]
        isem_d = [id0, id1]
        gsem_a = [ga0, ga1]
        gsem_b = [gb0, gb1]
        gsem_h = [gh0, gh1]
        ssem_e = [se0, se1]
        ssem_a = [sa0, sa1]

        def idx_start(c, p):
            base = wid * EW + c * EB
            pltpu.async_copy(src_hbm.at[pl.ds(base, EB)], idxb[p].at[0],
                             isem_s[p])
            pltpu.async_copy(dst_hbm.at[pl.ds(base, EB)], idxb[p].at[1],
                             isem_d[p])

        def idx_wait(p):
            pltpu.make_async_copy(src_hbm.at[pl.ds(0, EB)], idxb[p].at[0],
                                  isem_s[p]).wait()
            pltpu.make_async_copy(dst_hbm.at[pl.ds(0, EB)], idxb[p].at[1],
                                  isem_d[p]).wait()

        def gathers_start(p):
            pltpu.async_copy(asad_hbm.at[idxb[p].at[0]], asb[p], gsem_a[p])
            pltpu.async_copy(asad_hbm.at[idxb[p].at[1]], adb[p], gsem_b[p])
            pltpu.async_copy(h_hbm.at[idxb[p].at[0]], hb[p], gsem_h[p])

        def gathers_wait(p):
            pltpu.make_async_copy(asad_hbm.at[idxb[p].at[0]], asb[p],
                                  gsem_a[p]).wait()
            pltpu.make_async_copy(asad_hbm.at[idxb[p].at[1]], adb[p],
                                  gsem_b[p]).wait()
            pltpu.make_async_copy(h_hbm.at[idxb[p].at[0]], hb[p],
                                  gsem_h[p]).wait()

        def compute(p):
            m16 = m_v[...]
            for q in range(EB // 16):
                sdix[p][pl.ds(q * 16, 16)] = idxb[p][1, pl.ds(q * 16, 16)]

            @plsc.parallel_loop(0, EB, unroll=4)
            def _(j):
                s = asb[p][j, pl.ds(0, 16)] + adb[p][j, pl.ds(16, 16)]
                e = jnp.maximum(s, 0.2 * s)
                exv = jnp.exp(e - m16)
                exb[p][j, :] = exv
                for hh in range(H):
                    idx = jnp.full((16, 1), hh, jnp.int32)
                    scale = lax.gather(
                        exv, idx, _GDN, (1,),
                        mode=lax.GatherScatterMode.PROMISE_IN_BOUNDS)
                    hb[p][j, pl.ds(hh * DH, DH)] = (
                        hb[p][j, pl.ds(hh * DH, DH)] * scale)

            pltpu.async_copy(exb[p], den_sh.at[sdix[p]], ssem_e[p], add=True)
            pltpu.async_copy(hb[p], acc_sh.at[sdix[p]], ssem_a[p], add=True)

        def scat_wait(p):
            pltpu.make_async_copy(exb[p], den_sh.at[sdix[p]],
                                  ssem_e[p]).wait()
            pltpu.make_async_copy(hb[p], acc_sh.at[sdix[p]],
                                  ssem_a[p]).wait()

        @pl.loop(0, EB)
        def _(r):
            exb0[r, :] = zero16
            for cc in range(D // 16):
                hb0[r, pl.ds(cc * 16, 16)] = zero16

        r0 = sid * RPT
        for kk in range(RPT // EB):
            pltpu.sync_copy(hb0, acc_sh.at[pl.ds(r0 + kk * EB, EB), :])
            pltpu.sync_copy(exb0, den_sh.at[pl.ds(r0 + kk * EB, EB), :])

        pltpu.sync_copy(m_hbm.at[0, pl.ds(0, 16)], m_v)

        plsc.subcore_barrier()

        idx_start(0, 0)
        idx_wait(0)
        gathers_start(0)
        idx_start(1, 1)

        @pl.loop(0, NCHUNK - 1, step=2)
        def _(cbase):
            for p in (0, 1):
                c = cbase + p
                gathers_wait(p)

                @pl.when(c >= 1)
                def _():
                    scat_wait(1 - p)

                idx_wait(1 - p)
                gathers_start(1 - p)
                compute(p)

                @pl.when(c + 2 < NCHUNK)
                def _():
                    idx_start(c + 2, p)

        gathers_wait(0)
        scat_wait(1)
        compute(0)
        scat_wait(0)

        plsc.subcore_barrier()
        pltpu.sync_copy(acc_sh.at[pl.ds(r0, RPT), :],
                        acc_out.at[cid, pl.ds(r0, RPT), :])
        pltpu.sync_copy(den_sh.at[pl.ds(r0, RPT), :],
                        den_out.at[cid, pl.ds(r0, RPT), :])

    return k(h, asad_t, edge_index[0], edge_index[1], m_arr)


def _make_ssd(a_src, a_dst):
    eye = jnp.eye(H, dtype=jnp.float32)
    s = (a_src[:, :, None] * eye[:, None, :]).reshape(D, H)
    d_ = (a_dst[:, :, None] * eye[:, None, :]).reshape(D, H)
    z = jnp.zeros((D, H), jnp.float32)
    return jnp.concatenate([s, z, d_, z], axis=1)


def _make_p16():
    p = (jnp.eye(H, dtype=jnp.float32)[:, :, None]
         * jnp.ones((1, 1, DH), jnp.float32)).reshape(H, D)
    return jnp.concatenate([p, jnp.zeros((H, D), jnp.float32)], axis=0)


def kernel(X, edge_index, W0, a_src0, a_dst0, W1, a_src1, a_dst1,
           Wf1, bf1, Wf2, bf2, Wf3, bf3):
    ssd0 = _make_ssd(a_src0, a_dst0)
    ssd1 = _make_ssd(a_src1, a_dst1)
    p16 = _make_p16()
    h0, asad0, m0 = _prep0(X, W0, ssd0)
    accp0, denp0 = _edge_sc(h0, asad0, edge_index, m0)
    h1, asad1, m1 = _mid(accp0, denp0, p16, W1, ssd1)
    accp1, denp1 = _edge_sc(h1, asad1, edge_index, m1)
    return _final(accp1, denp1, p16, Wf1, bf1, Wf2, bf2, Wf3, bf3)


# speedup vs baseline: 1.0324x; 1.0324x over previous
import functools

import jax
import jax.numpy as jnp
from jax import lax
from jax.experimental import pallas as pl
from jax.experimental.pallas import tpu as pltpu
from jax.experimental.pallas import tpu_sc as plsc

N = 10000
E = 320000
D = 128
H = 8
DH = D // H
NC = 40

BLK = 1000
NGRID = N // BLK
NCORES = 2
NSUB = 16
NW = NCORES * NSUB
EW = E // NW
EB = 80
NCHUNK = EW // EB
NP = 10240
RPT = NP // NSUB


def _attn_tail(x, w_ref, ssd_ref, o_h, o_as, o_ad, o_m, ms):
    i = pl.program_id(0)
    h = jnp.dot(x, w_ref[...])
    o_h[...] = h
    asad = jnp.dot(h, ssd_ref[...])
    o_as[...] = asad[:, :16]
    o_ad[...] = asad[:, 16:]
    mx = jnp.max(asad, axis=0)

    @pl.when(i == 0)
    def _():
        ms[...] = jnp.full((8, 32), -1e30, jnp.float32)

    ms[...] = jnp.maximum(ms[...], mx[None, :])

    @pl.when(i == NGRID - 1)
    def _():
        m = ms[...]
        o_m[...] = jnp.maximum(m[:, :16] + m[:, 16:], 0.0)


def _combine(accp_ref, denp_ref, p16_ref):
    acc = accp_ref[0] + accp_ref[1]
    den = denp_ref[0] + denp_ref[1]
    rden = 1.0 / (den + 1e-16)
    rd = jnp.dot(rden, p16_ref[...])
    x = acc * rd
    return jnp.where(x > 0.0, x, jnp.exp(jnp.minimum(x, 0.0)) - 1.0)


def _prep0_body(x_ref, w_ref, ssd_ref, o_h, o_as, o_ad, o_m, ms):
    _attn_tail(x_ref[...], w_ref, ssd_ref, o_h, o_as, o_ad, o_m, ms)


def _mid_body(accp_ref, denp_ref, p16_ref, w_ref, ssd_ref,
              o_h, o_as, o_ad, o_m, ms):
    x = _combine(accp_ref, denp_ref, p16_ref)
    _attn_tail(x, w_ref, ssd_ref, o_h, o_as, o_ad, o_m, ms)


def _final_body(accp_ref, denp_ref, p16_ref, w1, b1, w2, b2, w3, b3, o_y):
    x = _combine(accp_ref, denp_ref, p16_ref)
    hdd = jnp.maximum(jnp.dot(x, w1[...]) + b1[...][None, :], 0.0)
    hdd = jnp.maximum(jnp.dot(hdd, w2[...]) + b2[...][None, :], 0.0)
    o_y[...] = jnp.dot(hdd, w3[...]) + b3[...][None, :]


def _prep0(X, W, ssd):
    return pl.pallas_call(
        _prep0_body,
        grid=(NGRID,),
        in_specs=[
            pl.BlockSpec((BLK, D), lambda i: (i, 0)),
            pl.BlockSpec((D, D), lambda i: (0, 0)),
            pl.BlockSpec((D, 32), lambda i: (0, 0)),
        ],
        out_specs=[
            pl.BlockSpec((BLK, D), lambda i: (i, 0)),
            pl.BlockSpec((BLK, 16), lambda i: (i, 0)),
            pl.BlockSpec((BLK, 16), lambda i: (i, 0)),
            pl.BlockSpec((8, 16), lambda i: (0, 0)),
        ],
        out_shape=[
            jax.ShapeDtypeStruct((N, D), jnp.float32),
            jax.ShapeDtypeStruct((N, 16), jnp.float32),
            jax.ShapeDtypeStruct((N, 16), jnp.float32),
            jax.ShapeDtypeStruct((8, 16), jnp.float32),
        ],
        scratch_shapes=[
            pltpu.VMEM((8, 32), jnp.float32),
        ],
    )(X, W, ssd)


def _mid(accp, denp, p16, W, ssd):
    return pl.pallas_call(
        _mid_body,
        grid=(NGRID,),
        in_specs=[
            pl.BlockSpec((2, BLK, D), lambda i: (0, i, 0)),
            pl.BlockSpec((2, BLK, 16), lambda i: (0, i, 0)),
            pl.BlockSpec((16, D), lambda i: (0, 0)),
            pl.BlockSpec((D, D), lambda i: (0, 0)),
            pl.BlockSpec((D, 32), lambda i: (0, 0)),
        ],
        out_specs=[
            pl.BlockSpec((BLK, D), lambda i: (i, 0)),
            pl.BlockSpec((BLK, 16), lambda i: (i, 0)),
            pl.BlockSpec((BLK, 16), lambda i: (i, 0)),
            pl.BlockSpec((8, 16), lambda i: (0, 0)),
        ],
        out_shape=[
            jax.ShapeDtypeStruct((N, D), jnp.float32),
            jax.ShapeDtypeStruct((N, 16), jnp.float32),
            jax.ShapeDtypeStruct((N, 16), jnp.float32),
            jax.ShapeDtypeStruct((8, 16), jnp.float32),
        ],
        scratch_shapes=[
            pltpu.VMEM((8, 32), jnp.float32),
        ],
    )(accp, denp, p16, W, ssd)


def _final(accp, denp, p16, Wf1, bf1, Wf2, bf2, Wf3, bf3):
    return pl.pallas_call(
        _final_body,
        grid=(NGRID,),
        in_specs=[
            pl.BlockSpec((2, BLK, D), lambda i: (0, i, 0)),
            pl.BlockSpec((2, BLK, 16), lambda i: (0, i, 0)),
            pl.BlockSpec((16, D), lambda i: (0, 0)),
            pl.BlockSpec((D, 512), lambda i: (0, 0)),
            pl.BlockSpec((512,), lambda i: (0,)),
            pl.BlockSpec((512, 512), lambda i: (0, 0)),
            pl.BlockSpec((512,), lambda i: (0,)),
            pl.BlockSpec((512, NC), lambda i: (0, 0)),
            pl.BlockSpec((NC,), lambda i: (0,)),
        ],
        out_specs=pl.BlockSpec((BLK, NC), lambda i: (i, 0)),
        out_shape=jax.ShapeDtypeStruct((N, NC), jnp.float32),
    )(accp, denp, p16, Wf1, bf1, Wf2, bf2, Wf3, bf3)


_GDN = lax.GatherDimensionNumbers(
    offset_dims=(), collapsed_slice_dims=(0,), start_index_map=(0,))

def _edge_sc(h, as_t, ad_t, edge_index, m_arr):
    mesh = plsc.VectorSubcoreMesh(core_axis_name="c", subcore_axis_name="s")

    @functools.partial(
        pl.kernel,
        mesh=mesh,
        compiler_params=pltpu.CompilerParams(use_tc_tiling_on_sc=False),
        out_type=[
            jax.ShapeDtypeStruct((NCORES, NP, D), jnp.float32),
            jax.ShapeDtypeStruct((NCORES, NP, 16), jnp.float32),
        ],
        scratch_types=[
            pltpu.VMEM_SHARED((NP, D), jnp.float32),
            pltpu.VMEM_SHARED((NP, 16), jnp.float32),
            pltpu.VMEM((2, EB), jnp.int32),
            pltpu.VMEM((2, EB), jnp.int32),
            pltpu.VMEM((EB, 16), jnp.float32),
            pltpu.VMEM((EB, 16), jnp.float32),
            pltpu.VMEM((EB, 16), jnp.float32),
            pltpu.VMEM((EB, 16), jnp.float32),
            pltpu.VMEM((EB, 16), jnp.float32),
            pltpu.VMEM((EB, 16), jnp.float32),
            pltpu.VMEM((EB, D), jnp.float32),
            pltpu.VMEM((EB, D), jnp.float32),
            pltpu.VMEM((EB,), jnp.int32),
            pltpu.VMEM((EB,), jnp.int32),
            pltpu.VMEM((16,), jnp.float32),
        ] + [pltpu.SemaphoreType.DMA] * 14,
    )
    def k(h_hbm, as_hbm, ad_hbm, src_hbm, dst_hbm, m_hbm, acc_out, den_out,
          acc_sh, den_sh, idxb0, idxb1, asb0, asb1, adb0, adb1,
          exb0, exb1, hb0, hb1, sdix0, sdix1, m_v,
          is0, is1, id0, id1, ga0, ga1, gb0, gb1, gh0, gh1,
          se0, se1, sa0, sa1):
        cid = lax.axis_index("c")
        sid = lax.axis_index("s")
        wid = cid * NSUB + sid
        zero16 = jnp.zeros((16,), jnp.float32)
        idxb = [idxb0, idxb1]
        asb = [asb0, asb1]
        adb = [adb0, adb1]
        exb = [exb0, exb1]
        hb = [hb0, hb1]
        sdix = [sdix0, sdix1]
        isem_s = [is0, is1]
        isem_d = [id0, id1]
        gsem_a = [ga0, ga1]
        gsem_b = [gb0, gb1]
        gsem_h = [gh0, gh1]
        ssem_e = [se0, se1]
        ssem_a = [sa0, sa1]

        def idx_start(c, p):
            base = wid * EW + c * EB
            pltpu.async_copy(src_hbm.at[pl.ds(base, EB)], idxb[p].at[0],
                             isem_s[p])
            pltpu.async_copy(dst_hbm.at[pl.ds(base, EB)], idxb[p].at[1],
                             isem_d[p])

        def idx_wait(p):
            pltpu.make_async_copy(src_hbm.at[pl.ds(0, EB)], idxb[p].at[0],
                                  isem_s[p]).wait()
            pltpu.make_async_copy(dst_hbm.at[pl.ds(0, EB)], idxb[p].at[1],
                                  isem_d[p]).wait()

        def gathers_start(p):
            pltpu.async_copy(as_hbm.at[idxb[p].at[0]], asb[p], gsem_a[p])
            pltpu.async_copy(ad_hbm.at[idxb[p].at[1]], adb[p], gsem_b[p])
            pltpu.async_copy(h_hbm.at[idxb[p].at[0]], hb[p], gsem_h[p])

        def gathers_wait(p):
            pltpu.make_async_copy(as_hbm.at[idxb[p].at[0]], asb[p],
                                  gsem_a[p]).wait()
            pltpu.make_async_copy(ad_hbm.at[idxb[p].at[1]], adb[p],
                                  gsem_b[p]).wait()
            pltpu.make_async_copy(h_hbm.at[idxb[p].at[0]], hb[p],
                                  gsem_h[p]).wait()

        def compute(p):
            m16 = m_v[...]
            for q in range(EB // 16):
                sdix[p][pl.ds(q * 16, 16)] = idxb[p][1, pl.ds(q * 16, 16)]

            @plsc.parallel_loop(0, EB, unroll=4)
            def _(j):
                s = asb[p][j, :] + adb[p][j, :]
                e = jnp.maximum(s, 0.2 * s)
                exv = jnp.exp(e - m16)
                exb[p][j, :] = exv
                for hh in range(H):
                    idx = jnp.full((16, 1), hh, jnp.int32)
                    scale = lax.gather(
                        exv, idx, _GDN, (1,),
                        mode=lax.GatherScatterMode.PROMISE_IN_BOUNDS)
                    hb[p][j, pl.ds(hh * DH, DH)] = (
                        hb[p][j, pl.ds(hh * DH, DH)] * scale)

            pltpu.async_copy(exb[p], den_sh.at[sdix[p]], ssem_e[p], add=True)
            pltpu.async_copy(hb[p], acc_sh.at[sdix[p]], ssem_a[p], add=True)

        def scat_wait(p):
            pltpu.make_async_copy(exb[p], den_sh.at[sdix[p]],
                                  ssem_e[p]).wait()
            pltpu.make_async_copy(hb[p], acc_sh.at[sdix[p]],
                                  ssem_a[p]).wait()

        @pl.loop(0, EB)
        def _(r):
            exb0[r, :] = zero16
            for cc in range(D // 16):
                hb0[r, pl.ds(cc * 16, 16)] = zero16

        r0 = sid * RPT
        for kk in range(RPT // EB):
            pltpu.sync_copy(hb0, acc_sh.at[pl.ds(r0 + kk * EB, EB), :])
            pltpu.sync_copy(exb0, den_sh.at[pl.ds(r0 + kk * EB, EB), :])

        pltpu.sync_copy(m_hbm.at[0, pl.ds(0, 16)], m_v)

        plsc.subcore_barrier()

        idx_start(0, 0)
        idx_wait(0)
        gathers_start(0)
        idx_start(1, 1)

        @pl.loop(0, NCHUNK - 1, step=2)
        def _(cbase):
            for p in (0, 1):
                c = cbase + p
                gathers_wait(p)

                @pl.when(c >= 1)
                def _():
                    scat_wait(1 - p)

                idx_wait(1 - p)
                gathers_start(1 - p)
                compute(p)

                @pl.when(c + 2 < NCHUNK)
                def _():
                    idx_start(c + 2, p)

        gathers_wait(0)
        scat_wait(1)
        compute(0)
        scat_wait(0)

        plsc.subcore_barrier()
        pltpu.sync_copy(acc_sh.at[pl.ds(r0, RPT), :],
                        acc_out.at[cid, pl.ds(r0, RPT), :])
        pltpu.sync_copy(den_sh.at[pl.ds(r0, RPT), :],
                        den_out.at[cid, pl.ds(r0, RPT), :])

    return k(h, as_t, ad_t, edge_index[0], edge_index[1], m_arr)


def _make_ssd(a_src, a_dst):
    eye = jnp.eye(H, dtype=jnp.float32)
    s = (a_src[:, :, None] * eye[:, None, :]).reshape(D, H)
    d_ = (a_dst[:, :, None] * eye[:, None, :]).reshape(D, H)
    z = jnp.zeros((D, H), jnp.float32)
    return jnp.concatenate([s, z, d_, z], axis=1)


def _make_p16():
    p = (jnp.eye(H, dtype=jnp.float32)[:, :, None]
         * jnp.ones((1, 1, DH), jnp.float32)).reshape(H, D)
    return jnp.concatenate([p, jnp.zeros((H, D), jnp.float32)], axis=0)


def kernel(X, edge_index, W0, a_src0, a_dst0, W1, a_src1, a_dst1,
           Wf1, bf1, Wf2, bf2, Wf3, bf3):
    ssd0 = _make_ssd(a_src0, a_dst0)
    ssd1 = _make_ssd(a_src1, a_dst1)
    p16 = _make_p16()
    h0, as0, ad0, m0 = _prep0(X, W0, ssd0)
    accp0, denp0 = _edge_sc(h0, as0, ad0, edge_index, m0)
    h1, as1, ad1, m1 = _mid(accp0, denp0, p16, W1, ssd1)
    accp1, denp1 = _edge_sc(h1, as1, ad1, edge_index, m1)
    return _final(accp1, denp1, p16, Wf1, bf1, Wf2, bf2, Wf3, bf3)
